# Initial kernel scaffold; baseline (speedup 1.0000x reference)
import jax
import jax.numpy as jnp
from jax.experimental import pallas as pl


def kernel(z, diffusion_constant, encoding_x, encoding_y, segment_ids):
    raise NotImplementedError("write your pallas kernel here")


# speedup vs baseline: 29.3842x; 29.3842x over previous
import functools

import jax
import jax.numpy as jnp
from jax.experimental import pallas as pl
from jax.experimental.pallas import tpu as pltpu

N_CELLS = 16384
N_SPOTS = 4096
N_LABELS = 512
BC = 512


def _body(zx_ref, zy_ref, ex_ref, ey_ref, seg_ref, d_ref, out_ref, onehot_ref):
    @pl.when(pl.program_id(0) == 0)
    def _():
        labs = jax.lax.broadcasted_iota(jnp.int32, (N_SPOTS, N_LABELS), 1)
        onehot_ref[...] = (seg_ref[...] == labs).astype(jnp.bfloat16)

    d = d_ref[0, 0]
    norm = 1.0 / (2.0 * jnp.pi * d)
    dx = ex_ref[...] - zx_ref[...]
    dy = ey_ref[...] - zy_ref[...]
    dist = dx * dx + dy * dy
    k = norm * jnp.exp(dist * (-0.5 / d)) + 1e-12
    out_ref[...] = jnp.dot(
        k.astype(jnp.bfloat16), onehot_ref[...],
        preferred_element_type=jnp.float32)


@jax.jit
def kernel(z, diffusion_constant, encoding_x, encoding_y, segment_ids):
    zx = z[:, 0:1]
    zy = z[:, 1:2]
    ex = encoding_x.reshape(1, N_SPOTS)
    ey = encoding_y.reshape(1, N_SPOTS)
    seg = segment_ids.reshape(N_SPOTS, 1)
    d = diffusion_constant.reshape(1, 1)

    grid = (N_CELLS // BC,)
    return pl.pallas_call(
        _body,
        grid=grid,
        in_specs=[
            pl.BlockSpec((BC, 1), lambda i: (i, 0)),
            pl.BlockSpec((BC, 1), lambda i: (i, 0)),
            pl.BlockSpec((1, N_SPOTS), lambda i: (0, 0)),
            pl.BlockSpec((1, N_SPOTS), lambda i: (0, 0)),
            pl.BlockSpec((N_SPOTS, 1), lambda i: (0, 0)),
            pl.BlockSpec((1, 1), lambda i: (0, 0)),
        ],
        out_specs=pl.BlockSpec((BC, N_LABELS), lambda i: (i, 0)),
        out_shape=jax.ShapeDtypeStruct((N_CELLS, N_LABELS), jnp.float32),
        scratch_shapes=[pltpu.VMEM((N_SPOTS, N_LABELS), jnp.bfloat16)],
        compiler_params=pltpu.CompilerParams(
            dimension_semantics=("arbitrary",),
        ),
    )(zx, zy, ex, ey, seg, d)


# speedup vs baseline: 29.8350x; 1.0153x over previous
import math

import jax
import jax.numpy as jnp
from jax.experimental import pallas as pl
from jax.experimental.pallas import tpu as pltpu

N_CELLS = 16384
N_SPOTS = 4096
N_LABELS = 512
BC = 1024
CHUNK = 512
LOG2E = math.log2(math.e)


def _body(ax_ref, ay_ref, col_ref, ex_ref, ey_ref, row_ref, onehot_ref,
          out_ref):
    ax = ax_ref[...]
    ay = ay_ref[...]
    col = col_ref[...]
    acc = None
    for c in range(N_SPOTS // CHUNK):
        sl = pl.ds(c * CHUNK, CHUNK)
        t = (ax * ex_ref[:, sl] + ay * ey_ref[:, sl]) + (row_ref[:, sl] + col)
        k = (jnp.exp2(t) + 1e-12).astype(jnp.bfloat16)
        p = jnp.dot(k, onehot_ref[sl, :], preferred_element_type=jnp.float32)
        acc = p if acc is None else acc + p
    out_ref[...] = acc


@jax.jit
def kernel(z, diffusion_constant, encoding_x, encoding_y, segment_ids):
    d = diffusion_constant.astype(jnp.float32)
    c1 = -LOG2E / (2.0 * d)
    c2 = -jnp.log2(2.0 * jnp.pi * d)

    zx = z[:, 0:1]
    zy = z[:, 1:2]
    ax = (-2.0 * c1) * zx
    ay = (-2.0 * c1) * zy
    col = c1 * (zx * zx + zy * zy)
    ex = encoding_x.reshape(1, N_SPOTS)
    ey = encoding_y.reshape(1, N_SPOTS)
    row = c1 * (ex * ex + ey * ey) + c2
    onehot = (segment_ids[:, None]
              == jnp.arange(N_LABELS, dtype=jnp.int32)[None, :]
              ).astype(jnp.bfloat16)

    grid = (N_CELLS // BC,)
    return pl.pallas_call(
        _body,
        grid=grid,
        in_specs=[
            pl.BlockSpec((BC, 1), lambda i: (i, 0)),
            pl.BlockSpec((BC, 1), lambda i: (i, 0)),
            pl.BlockSpec((BC, 1), lambda i: (i, 0)),
            pl.BlockSpec((1, N_SPOTS), lambda i: (0, 0)),
            pl.BlockSpec((1, N_SPOTS), lambda i: (0, 0)),
            pl.BlockSpec((1, N_SPOTS), lambda i: (0, 0)),
            pl.BlockSpec((N_SPOTS, N_LABELS), lambda i: (0, 0)),
        ],
        out_specs=pl.BlockSpec((BC, N_LABELS), lambda i: (i, 0)),
        out_shape=jax.ShapeDtypeStruct((N_CELLS, N_LABELS), jnp.float32),
        compiler_params=pltpu.CompilerParams(
            dimension_semantics=("arbitrary",),
        ),
    )(ax, ay, col, ex, ey, row, onehot)
